# baseline (device time: 147302 ns/iter reference)
import jax
import jax.numpy as jnp
from jax import lax
from jax.experimental import pallas as pl
from jax.experimental.pallas import tpu as pltpu

N_DEV = 4


def _local_partial(x, Wq, K_ext, V_ext, Wo, my):
    B, Sq, D = x.shape
    Hl, Dh = K_ext.shape[2:]
    HD = Hl * Dh

    Wq_loc = lax.dynamic_slice(Wq, (0, my * HD), (D, HD))
    Wo_loc = lax.dynamic_slice(Wo, (my * HD, 0), (HD, D))

    Q = (x @ Wq_loc).reshape(B, Sq, Hl, Dh)

    def group(t):
        return (
            t.reshape(B, 2, 4, 64, Hl, Dh)
            .transpose(0, 2, 1, 3, 4, 5)
            .reshape(B, 4, 128, Hl, Dh)
        )

    Qg, Kg, Vg = group(Q), group(K_ext), group(V_ext)
    scores = jnp.einsum("bgihd,bgjhd->bghij", Qg, Kg) * 0.125
    w = jax.nn.softmax(scores, axis=-1)
    ctx_g = jnp.einsum("bghij,bgjhd->bgihd", w, Vg)
    ctx = (
        ctx_g.reshape(B, 4, 2, 64, Hl, Dh)
        .transpose(0, 2, 1, 3, 4, 5)
        .reshape(B, Sq, HD)
    )
    return ctx @ Wo_loc


def _ring_allreduce(partial):
    B, S, D = partial.shape

    def body(p_ref, out_ref, comm_ref, send_sems, recv_sems):
        my = lax.axis_index("i")
        left = lax.rem(my + N_DEV - 1, N_DEV)
        right = lax.rem(my + 1, N_DEV)

        barrier = pltpu.get_barrier_semaphore()
        for nbr in (left, right):
            pl.semaphore_signal(
                barrier, inc=1,
                device_id=(nbr,), device_id_type=pl.DeviceIdType.MESH,
            )
        pl.semaphore_wait(barrier, 2)

        out_ref[...] = p_ref[...]
        for h in range(N_DEV - 1):
            src = p_ref if h == 0 else comm_ref.at[h - 1]
            rdma = pltpu.make_async_remote_copy(
                src_ref=src,
                dst_ref=comm_ref.at[h],
                send_sem=send_sems.at[h],
                recv_sem=recv_sems.at[h],
                device_id=(right,),
                device_id_type=pl.DeviceIdType.MESH,
            )
            rdma.start()
            rdma.wait()
            out_ref[...] += comm_ref[h]

    return pl.pallas_call(
        body,
        out_shape=jax.ShapeDtypeStruct((B, S, D), partial.dtype),
        in_specs=[pl.BlockSpec(memory_space=pltpu.VMEM)],
        out_specs=pl.BlockSpec(memory_space=pltpu.VMEM),
        scratch_shapes=[
            pltpu.VMEM((N_DEV - 1, B, S, D), partial.dtype),
            pltpu.SemaphoreType.DMA((N_DEV - 1,)),
            pltpu.SemaphoreType.DMA((N_DEV - 1,)),
        ],
        compiler_params=pltpu.CompilerParams(collective_id=0),
    )(partial)


def kernel(x, Wq, K_ext, V_ext, Wo):
    my = lax.axis_index("i")
    partial = _local_partial(x, Wq, K_ext, V_ext, Wo, my)
    return _ring_allreduce(partial)
